# Initial kernel scaffold; baseline (speedup 1.0000x reference)
import jax
import jax.numpy as jnp
from jax.experimental import pallas as pl


def kernel(x, edge_index, edge_weight, label_edge_index, W1, b1, a1, W2, b2, a2, Wd1, bd1, ad, Wd2, bd2):
    raise NotImplementedError("write your pallas kernel here")


# speedup vs baseline: 8.8714x; 8.8714x over previous
import functools

import jax
import jax.numpy as jnp
from jax import lax
from jax.experimental import pallas as pl
from jax.experimental.pallas import tpu as pltpu
from jax.experimental.pallas import tpu_sc as plsc

N = 10000
NPAD = 10240
E = 320000
EPAD = 323584
NL = 100000
NLPAD = 102400
F_IN = 128
C = 64

NC = 2
NS = 16
NW = NC * NS

EPT_A = EPAD // NS
NVEC_A = EPT_A // 16
DROWS = NPAD // 16
DROWS_PT = DROWS // NS

EPT_B = EPAD // NW
BCH = 128
NCH_B = EPT_B // BCH
ROWS_PT = NPAD // NS

LPT = NLPAD // NW
NCH_D = LPT // BCH

_mesh = plsc.VectorSubcoreMesh(
    core_axis_name="c", subcore_axis_name="s", num_cores=NC, num_subcores=NS)
_sc_params = pltpu.CompilerParams(needs_layout_passes=False,
                                  use_tc_tiling_on_sc=False)


def _rsqrt_newton(d):
  i = plsc.bitcast(d, jnp.int32)
  i = jnp.int32(0x5F3759DF) - lax.shift_right_logical(i, 1)
  z = plsc.bitcast(i, jnp.float32)
  half = d * 0.5
  for _ in range(3):
    z = z * (1.5 - half * z * z)
  return z


def _sc_deg_body(src_hbm, dst_hbm, ew_hbm, zdeg_hbm, iota_hbm,
                 dis_hbm, w_hbm,
                 src_v, dst_v, ew_v, part_v, idx_v, tmp_v, w_v, acc_ref, sem):
  c = lax.axis_index("c")
  s = lax.axis_index("s")

  @pl.when(c == 0)
  def _():
    base = s * EPT_A
    pltpu.sync_copy(dst_hbm.at[pl.ds(base, EPT_A)], dst_v)
    pltpu.sync_copy(ew_hbm.at[pl.ds(base, EPT_A)], ew_v)
    pltpu.sync_copy(zdeg_hbm, part_v)
    pltpu.sync_copy(iota_hbm, idx_v)
    pltpu.sync_copy(zdeg_hbm.at[pl.ds(s * DROWS_PT, DROWS_PT)],
                    acc_ref.at[pl.ds(s * DROWS_PT, DROWS_PT)])

    @pl.loop(0, NVEC_A)
    def _(i):
      d16 = dst_v[pl.ds(i * 16, 16)]
      e16 = ew_v[pl.ds(i * 16, 16)]
      plsc.addupdate_scatter(
          part_v,
          [lax.shift_right_logical(d16, 4), jnp.bitwise_and(d16, 15)], e16)

    plsc.subcore_barrier()
    @pl.loop(0, DROWS // BCH)
    def _(j):
      pltpu.async_copy(part_v.at[pl.ds(j * BCH, BCH)],
                       acc_ref.at[idx_v.at[j]], sem, add=True).wait()
    plsc.subcore_barrier()

    rbase = s * DROWS_PT
    pltpu.sync_copy(acc_ref.at[pl.ds(rbase, DROWS_PT)], tmp_v)

    @pl.loop(0, DROWS_PT)
    def _(r):
      tmp_v[r] = _rsqrt_newton(tmp_v[r] + 1.0)

    pltpu.sync_copy(tmp_v, acc_ref.at[pl.ds(rbase, DROWS_PT)])
    pltpu.sync_copy(tmp_v, dis_hbm.at[pl.ds(rbase, DROWS_PT)])
    plsc.subcore_barrier()
    pltpu.sync_copy(acc_ref, part_v)

    pltpu.sync_copy(src_hbm.at[pl.ds(base, EPT_A)], src_v)

    @pl.loop(0, NVEC_A)
    def _(i):
      s16 = src_v[pl.ds(i * 16, 16)]
      d16 = plsc.load_gather(
          part_v,
          [lax.shift_right_logical(s16, 4), jnp.bitwise_and(s16, 15)])
      w_v[pl.ds(i * 16, 16)] = d16 * ew_v[pl.ds(i * 16, 16)]

    pltpu.sync_copy(w_v, w_hbm.at[pl.ds(base, EPT_A)])


_sc_deg = pl.kernel(
    _sc_deg_body,
    out_type=[jax.ShapeDtypeStruct((DROWS, 16), jnp.float32),
              jax.ShapeDtypeStruct((EPAD,), jnp.float32)],
    mesh=_mesh,
    compiler_params=_sc_params,
    scratch_types=[
        pltpu.VMEM((EPT_A,), jnp.int32),
        pltpu.VMEM((EPT_A,), jnp.int32),
        pltpu.VMEM((EPT_A,), jnp.float32),
        pltpu.VMEM((DROWS, 16), jnp.float32),
        pltpu.VMEM((DROWS // BCH, BCH), jnp.int32),
        pltpu.VMEM((DROWS_PT, 16), jnp.float32),
        pltpu.VMEM((EPT_A,), jnp.float32),
        pltpu.VMEM_SHARED((DROWS, 16), jnp.float32),
        pltpu.SemaphoreType.DMA,
    ])


def _sc_scatter_body(h_hbm, src_hbm, dst_hbm, w_hbm, zrows_hbm,
                     out_hbm,
                     src_cv, dst_cv, w_cv, rows_v, acc_ref, gsem, ssem):
  c = lax.axis_index("c")
  s = lax.axis_index("s")
  wid = c * NS + s

  pltpu.sync_copy(zrows_hbm, acc_ref.at[pl.ds(s * ROWS_PT, ROWS_PT)])
  plsc.subcore_barrier()

  base = wid * EPT_B

  @pl.loop(0, NCH_B)
  def _(i):
    eb = base + i * BCH
    pltpu.sync_copy(src_hbm.at[pl.ds(eb, BCH)], src_cv)
    pltpu.sync_copy(dst_hbm.at[pl.ds(eb, BCH)], dst_cv)
    pltpu.sync_copy(w_hbm.at[pl.ds(eb, BCH)], w_cv)
    pltpu.async_copy(h_hbm.at[src_cv], rows_v, gsem).wait()

    @pl.loop(0, BCH)
    def _(e):
      wv = plsc.load_gather(w_cv, [jnp.full((16,), e, jnp.int32)])
      for q in range(4):
        sl = pl.ds(q * 16, 16)
        rows_v[e, sl] = rows_v[e, sl] * wv

    pltpu.async_copy(rows_v, acc_ref.at[dst_cv], ssem, add=True).wait()

  plsc.subcore_barrier()
  pltpu.sync_copy(acc_ref.at[pl.ds(s * ROWS_PT, ROWS_PT)],
                  out_hbm.at[c, pl.ds(s * ROWS_PT, ROWS_PT)])


_sc_scatter = pl.kernel(
    _sc_scatter_body,
    out_type=jax.ShapeDtypeStruct((NC, NPAD, C), jnp.float32),
    mesh=_mesh,
    compiler_params=_sc_params,
    scratch_types=[
        pltpu.VMEM((BCH,), jnp.int32),
        pltpu.VMEM((BCH,), jnp.int32),
        pltpu.VMEM((BCH,), jnp.float32),
        pltpu.VMEM((BCH, C), jnp.float32),
        pltpu.VMEM_SHARED((NPAD, C), jnp.float32),
        pltpu.SemaphoreType.DMA,
        pltpu.SemaphoreType.DMA,
    ])


def _sc_pairs_body(a_hbm, b_hbm, l0_hbm, l1_hbm,
                   zp_hbm,
                   l0_v, l1_v, ra_v, rb_v, sa, sb):
  c = lax.axis_index("c")
  s = lax.axis_index("s")
  wid = c * NS + s
  base = wid * LPT
  pltpu.sync_copy(l0_hbm.at[pl.ds(base, LPT)], l0_v)
  pltpu.sync_copy(l1_hbm.at[pl.ds(base, LPT)], l1_v)

  @pl.loop(0, NCH_D)
  def _(i):
    cpa = pltpu.async_copy(a_hbm.at[l0_v.at[pl.ds(i * BCH, BCH)]], ra_v, sa)
    cpb = pltpu.async_copy(b_hbm.at[l1_v.at[pl.ds(i * BCH, BCH)]], rb_v, sb)
    cpa.wait()
    cpb.wait()

    @pl.loop(0, BCH)
    def _(r):
      for q in range(4):
        sl = pl.ds(q * 16, 16)
        ra_v[r, sl] = ra_v[r, sl] + rb_v[r, sl]

    pltpu.sync_copy(ra_v, zp_hbm.at[pl.ds(base + i * BCH, BCH)])


_sc_pairs = pl.kernel(
    _sc_pairs_body,
    out_type=jax.ShapeDtypeStruct((NLPAD, C), jnp.float32),
    mesh=_mesh,
    compiler_params=_sc_params,
    scratch_types=[
        pltpu.VMEM((LPT,), jnp.int32),
        pltpu.VMEM((LPT,), jnp.int32),
        pltpu.VMEM((BCH, C), jnp.float32),
        pltpu.VMEM((BCH, C), jnp.float32),
        pltpu.SemaphoreType.DMA,
        pltpu.SemaphoreType.DMA,
    ])


_DOT = functools.partial(jnp.dot, precision=lax.Precision.HIGHEST)
_RB = 1000


def _tc_mm1_body(x_ref, w_ref, o_ref):
  o_ref[...] = _DOT(x_ref[...], w_ref[...])


def _tc_mm1(x, w1):
  return pl.pallas_call(
      _tc_mm1_body,
      grid=(N // _RB,),
      in_specs=[pl.BlockSpec((_RB, F_IN), lambda i: (i, 0)),
                pl.BlockSpec((F_IN, C), lambda i: (0, 0))],
      out_specs=pl.BlockSpec((_RB, C), lambda i: (i, 0)),
      out_shape=jax.ShapeDtypeStruct((N, C), jnp.float32),
  )(x, w1)


def _tc_mid_body(acc_ref, hp_ref, dis_ref, b_ref, a_ref, w_ref, o_ref):
  dis = dis_ref[...]
  acc = acc_ref[0] + acc_ref[1]
  pre = (acc + dis * hp_ref[...]) * dis + b_ref[...]
  h = jnp.where(pre >= 0, pre, a_ref[0, 0] * pre)
  o_ref[...] = _DOT(h, w_ref[...])


def _tc_mid(acc, hp, dis, b, a, w):
  return pl.pallas_call(
      _tc_mid_body,
      grid=(N // _RB,),
      in_specs=[pl.BlockSpec((NC, _RB, C), lambda i: (0, i, 0)),
                pl.BlockSpec((_RB, C), lambda i: (i, 0)),
                pl.BlockSpec((_RB, 1), lambda i: (i, 0)),
                pl.BlockSpec((1, C), lambda i: (0, 0)),
                pl.BlockSpec((1, 1), lambda i: (0, 0)),
                pl.BlockSpec((C, C), lambda i: (0, 0))],
      out_specs=pl.BlockSpec((_RB, C), lambda i: (i, 0)),
      out_shape=jax.ShapeDtypeStruct((N, C), jnp.float32),
  )(acc, hp, dis, b, a, w)


def _tc_head_body(acc_ref, hp_ref, dis_ref, b_ref, a_ref, wa_ref, wb_ref,
                  bd_ref, oa_ref, ob_ref):
  dis = dis_ref[...]
  acc = acc_ref[0] + acc_ref[1]
  pre = (acc + dis * hp_ref[...]) * dis + b_ref[...]
  h = jnp.where(pre >= 0, pre, a_ref[0, 0] * pre)
  oa_ref[...] = _DOT(h, wa_ref[...]) + bd_ref[...]
  ob_ref[...] = _DOT(h, wb_ref[...])


def _tc_head(acc, hp, dis, b, a, wa, wb, bd):
  return pl.pallas_call(
      _tc_head_body,
      grid=(N // _RB,),
      in_specs=[pl.BlockSpec((NC, _RB, C), lambda i: (0, i, 0)),
                pl.BlockSpec((_RB, C), lambda i: (i, 0)),
                pl.BlockSpec((_RB, 1), lambda i: (i, 0)),
                pl.BlockSpec((1, C), lambda i: (0, 0)),
                pl.BlockSpec((1, 1), lambda i: (0, 0)),
                pl.BlockSpec((C, C), lambda i: (0, 0)),
                pl.BlockSpec((C, C), lambda i: (0, 0)),
                pl.BlockSpec((1, C), lambda i: (0, 0))],
      out_specs=[pl.BlockSpec((_RB, C), lambda i: (i, 0)),
                 pl.BlockSpec((_RB, C), lambda i: (i, 0))],
      out_shape=[jax.ShapeDtypeStruct((N, C), jnp.float32),
                 jax.ShapeDtypeStruct((N, C), jnp.float32)],
  )(acc, hp, dis, b, a, wa, wb, bd)


_LB = 6400


def _tc_out_body(zp_ref, a_ref, w_ref, b_ref, o_ref):
  z = zp_ref[...]
  z = jnp.where(z >= 0, z, a_ref[0, 0] * z)
  o_ref[...] = _DOT(z, w_ref[...]) + b_ref[0, 0]


def _tc_out(zp, a, w, b):
  return pl.pallas_call(
      _tc_out_body,
      grid=(NLPAD // _LB,),
      in_specs=[pl.BlockSpec((_LB, C), lambda i: (i, 0)),
                pl.BlockSpec((1, 1), lambda i: (0, 0)),
                pl.BlockSpec((C, 1), lambda i: (0, 0)),
                pl.BlockSpec((1, 1), lambda i: (0, 0))],
      out_specs=pl.BlockSpec((_LB, 1), lambda i: (i, 0)),
      out_shape=jax.ShapeDtypeStruct((NLPAD, 1), jnp.float32),
  )(zp, a, w, b)


def kernel(x, edge_index, edge_weight, label_edge_index,
           W1, b1, a1, W2, b2, a2, Wd1, bd1, ad, Wd2, bd2):
  i32 = jnp.int32
  f32 = jnp.float32
  src = edge_index[0].astype(i32)
  dst = edge_index[1].astype(i32)
  ew = edge_weight.astype(f32)
  epad = EPAD - E
  src_p = jnp.concatenate([src, jnp.zeros((epad,), i32)])
  dst_p = jnp.concatenate([dst, jnp.zeros((epad,), i32)])
  ew_p = jnp.concatenate([ew, jnp.zeros((epad,), f32)])
  lpad = NLPAD - NL
  l0_p = jnp.concatenate([label_edge_index[0].astype(i32),
                          jnp.zeros((lpad,), i32)])
  l1_p = jnp.concatenate([label_edge_index[1].astype(i32),
                          jnp.zeros((lpad,), i32)])

  zdeg = jnp.zeros((DROWS, 16), f32)
  iota_rows = jnp.arange(DROWS, dtype=i32).reshape(DROWS // BCH, BCH)
  zrows = jnp.zeros((ROWS_PT, C), f32)

  dis2d, w_e = _sc_deg(src_p, dst_p, ew_p, zdeg, iota_rows)
  dis = dis2d.reshape(NPAD)[:N].reshape(N, 1)

  h1p = _tc_mm1(x, W1)
  acc1 = _sc_scatter(h1p, src_p, dst_p, w_e, zrows)
  h2p = _tc_mid(acc1, h1p, dis, b1.reshape(1, C), a1.reshape(1, 1), W2)

  acc2 = _sc_scatter(h2p, src_p, dst_p, w_e, zrows)
  A, B = _tc_head(acc2, h2p, dis, b2.reshape(1, C), a2.reshape(1, 1),
                  Wd1[:C], Wd1[C:], bd1.reshape(1, C))

  zp = _sc_pairs(A, B, l0_p, l1_p)
  out = _tc_out(zp, ad.reshape(1, 1), Wd2, bd2.reshape(1, 1))
  return out[:NL]


# speedup vs baseline: 14.1956x; 1.6002x over previous
import functools

import jax
import jax.numpy as jnp
from jax import lax
from jax.experimental import pallas as pl
from jax.experimental.pallas import tpu as pltpu
from jax.experimental.pallas import tpu_sc as plsc

N = 10000
NPAD = 10240
E = 320000
EPAD = 323584
NL = 100000
NLPAD = 102400
F_IN = 128
C = 64

NC = 2
NS = 16
NW = NC * NS

EPT_A = EPAD // NS
NVEC_A = EPT_A // 16
DROWS = NPAD // 16
DROWS_PT = DROWS // NS

EPT_B = EPAD // NW
BCH = 128
NCH_B = EPT_B // BCH
ROWS_PT = NPAD // NS

LPT = NLPAD // NW
NCH_D = LPT // BCH

_mesh = plsc.VectorSubcoreMesh(
    core_axis_name="c", subcore_axis_name="s", num_cores=NC, num_subcores=NS)
_sc_params = pltpu.CompilerParams(needs_layout_passes=False,
                                  use_tc_tiling_on_sc=False)


def _rsqrt_newton(d):
  i = plsc.bitcast(d, jnp.int32)
  i = jnp.int32(0x5F3759DF) - lax.shift_right_logical(i, 1)
  z = plsc.bitcast(i, jnp.float32)
  half = d * 0.5
  for _ in range(3):
    z = z * (1.5 - half * z * z)
  return z


def _sc_deg_body(src_hbm, dst_hbm, ew_hbm, zdeg_hbm, iota_hbm,
                 dis_hbm, w_hbm,
                 src_v, dst_v, ew_v, part_v, idx_v, tmp_v, w_v, acc_ref, sem):
  c = lax.axis_index("c")
  s = lax.axis_index("s")

  @pl.when(c == 0)
  def _():
    base = s * EPT_A
    pltpu.sync_copy(dst_hbm.at[pl.ds(base, EPT_A)], dst_v)
    pltpu.sync_copy(ew_hbm.at[pl.ds(base, EPT_A)], ew_v)
    pltpu.sync_copy(zdeg_hbm, part_v)
    pltpu.sync_copy(iota_hbm, idx_v)
    pltpu.sync_copy(zdeg_hbm.at[pl.ds(s * DROWS_PT, DROWS_PT)],
                    acc_ref.at[pl.ds(s * DROWS_PT, DROWS_PT)])

    @pl.loop(0, NVEC_A)
    def _(i):
      d16 = dst_v[pl.ds(i * 16, 16)]
      e16 = ew_v[pl.ds(i * 16, 16)]
      plsc.addupdate_scatter(
          part_v,
          [lax.shift_right_logical(d16, 4), jnp.bitwise_and(d16, 15)], e16)

    plsc.subcore_barrier()
    @pl.loop(0, DROWS // BCH)
    def _(j):
      pltpu.async_copy(part_v.at[pl.ds(j * BCH, BCH)],
                       acc_ref.at[idx_v.at[j]], sem, add=True).wait()
    plsc.subcore_barrier()

    rbase = s * DROWS_PT
    pltpu.sync_copy(acc_ref.at[pl.ds(rbase, DROWS_PT)], tmp_v)

    @pl.loop(0, DROWS_PT)
    def _(r):
      tmp_v[r] = _rsqrt_newton(tmp_v[r] + 1.0)

    pltpu.sync_copy(tmp_v, acc_ref.at[pl.ds(rbase, DROWS_PT)])
    pltpu.sync_copy(tmp_v, dis_hbm.at[pl.ds(rbase, DROWS_PT)])
    plsc.subcore_barrier()
    pltpu.sync_copy(acc_ref, part_v)

    pltpu.sync_copy(src_hbm.at[pl.ds(base, EPT_A)], src_v)

    @pl.loop(0, NVEC_A)
    def _(i):
      s16 = src_v[pl.ds(i * 16, 16)]
      d16 = plsc.load_gather(
          part_v,
          [lax.shift_right_logical(s16, 4), jnp.bitwise_and(s16, 15)])
      w_v[pl.ds(i * 16, 16)] = d16 * ew_v[pl.ds(i * 16, 16)]

    pltpu.sync_copy(w_v, w_hbm.at[pl.ds(base, EPT_A)])


_sc_deg = pl.kernel(
    _sc_deg_body,
    out_type=[jax.ShapeDtypeStruct((DROWS, 16), jnp.float32),
              jax.ShapeDtypeStruct((EPAD,), jnp.float32)],
    mesh=_mesh,
    compiler_params=_sc_params,
    scratch_types=[
        pltpu.VMEM((EPT_A,), jnp.int32),
        pltpu.VMEM((EPT_A,), jnp.int32),
        pltpu.VMEM((EPT_A,), jnp.float32),
        pltpu.VMEM((DROWS, 16), jnp.float32),
        pltpu.VMEM((DROWS // BCH, BCH), jnp.int32),
        pltpu.VMEM((DROWS_PT, 16), jnp.float32),
        pltpu.VMEM((EPT_A,), jnp.float32),
        pltpu.VMEM_SHARED((DROWS, 16), jnp.float32),
        pltpu.SemaphoreType.DMA,
    ])


def _sc_scatter_body(h_hbm, src_hbm, dst2_hbm, w_hbm, zrows_hbm,
                     out_hbm,
                     src_v, dst_v, w_v, r0, r1, r2, r3,
                     acc_ref, g0, g1, g2, g3, s0, s1, s2, s3):
  c = lax.axis_index("c")
  s = lax.axis_index("s")
  wid = c * NS + s
  base = wid * EPT_B
  rows = [r0, r1, r2, r3]
  gs = [g0, g1, g2, g3]
  ss = [s0, s1, s2, s3]

  pltpu.sync_copy(src_hbm.at[pl.ds(base, EPT_B)], src_v)
  pltpu.sync_copy(dst2_hbm.at[wid], dst_v)
  pltpu.sync_copy(w_hbm.at[pl.ds(base, EPT_B)], w_v)
  pltpu.sync_copy(zrows_hbm, acc_ref.at[pl.ds(s * ROWS_PT, ROWS_PT)])
  plsc.subcore_barrier()

  def gather_start(ch, b):
    pltpu.async_copy(h_hbm.at[src_v.at[pl.ds(ch * BCH, BCH)]], rows[b], gs[b])

  def gather_wait(b):
    pltpu.make_async_copy(h_hbm.at[pl.ds(0, BCH)], rows[b], gs[b]).wait()

  def scatter_start(ch, b):
    pltpu.async_copy(rows[b], acc_ref.at[dst_v.at[ch]], ss[b], add=True)

  def scatter_wait(b):
    pltpu.make_async_copy(h_hbm.at[pl.ds(0, BCH)], rows[b], ss[b]).wait()

  def scale(ch, b):
    rb = rows[b]

    @pl.loop(0, BCH)
    def _(e):
      wv = plsc.load_gather(w_v, [jnp.full((16,), ch * BCH + e, jnp.int32)])
      for q in range(4):
        sl = pl.ds(q * 16, 16)
        rb[e, sl] = rb[e, sl] * wv

  gather_start(0, 0)

  @pl.loop(0, (NCH_B - 3) // 4)
  def _(j):
    for b in range(4):
      i = 4 * j + b
      bn = (b + 1) % 4
      if b < 3:
        @pl.when(j > 0)
        def _():
          scatter_wait(bn)
      else:
        scatter_wait(bn)
      gather_start(i + 1, bn)
      gather_wait(b)
      scale(i, b)
      scatter_start(i, b)

  for i in (NCH_B - 3, NCH_B - 2, NCH_B - 1):
    b = i % 4
    bn = (b + 1) % 4
    if i + 1 < NCH_B:
      scatter_wait(bn)
      gather_start(i + 1, bn)
    gather_wait(b)
    scale(i, b)
    scatter_start(i, b)
  scatter_wait(3)
  scatter_wait(0)
  scatter_wait(1)
  scatter_wait(2)

  plsc.subcore_barrier()
  pltpu.sync_copy(acc_ref.at[pl.ds(s * ROWS_PT, ROWS_PT)],
                  out_hbm.at[c, pl.ds(s * ROWS_PT, ROWS_PT)])


_sc_scatter = pl.kernel(
    _sc_scatter_body,
    out_type=jax.ShapeDtypeStruct((NC, NPAD, C), jnp.float32),
    mesh=_mesh,
    compiler_params=_sc_params,
    scratch_types=[
        pltpu.VMEM((EPT_B,), jnp.int32),
        pltpu.VMEM((NCH_B, BCH), jnp.int32),
        pltpu.VMEM((EPT_B,), jnp.float32),
        pltpu.VMEM((BCH, C), jnp.float32),
        pltpu.VMEM((BCH, C), jnp.float32),
        pltpu.VMEM((BCH, C), jnp.float32),
        pltpu.VMEM((BCH, C), jnp.float32),
        pltpu.VMEM_SHARED((NPAD, C), jnp.float32),
        pltpu.SemaphoreType.DMA,
        pltpu.SemaphoreType.DMA,
        pltpu.SemaphoreType.DMA,
        pltpu.SemaphoreType.DMA,
        pltpu.SemaphoreType.DMA,
        pltpu.SemaphoreType.DMA,
        pltpu.SemaphoreType.DMA,
        pltpu.SemaphoreType.DMA,
    ])


def _sc_pairs_body(a_hbm, b_hbm, l0_hbm, l1_hbm,
                   zp_hbm,
                   l0_v, l1_v, ra0, ra1, rb0, rb1,
                   ga0, ga1, gb0, gb1, os0, os1):
  c = lax.axis_index("c")
  s = lax.axis_index("s")
  wid = c * NS + s
  base = wid * LPT
  ra = [ra0, ra1]
  rb = [rb0, rb1]
  ga = [ga0, ga1]
  gb = [gb0, gb1]
  os_ = [os0, os1]
  pltpu.sync_copy(l0_hbm.at[pl.ds(base, LPT)], l0_v)
  pltpu.sync_copy(l1_hbm.at[pl.ds(base, LPT)], l1_v)

  def gathers_start(ch, b):
    pltpu.async_copy(a_hbm.at[l0_v.at[pl.ds(ch * BCH, BCH)]], ra[b], ga[b])
    pltpu.async_copy(b_hbm.at[l1_v.at[pl.ds(ch * BCH, BCH)]], rb[b], gb[b])

  def gathers_wait(b):
    pltpu.make_async_copy(a_hbm.at[pl.ds(0, BCH)], ra[b], ga[b]).wait()
    pltpu.make_async_copy(b_hbm.at[pl.ds(0, BCH)], rb[b], gb[b]).wait()

  def out_start(ch, b):
    pltpu.async_copy(ra[b], zp_hbm.at[pl.ds(base + ch * BCH, BCH)], os_[b])

  def out_wait(b):
    pltpu.make_async_copy(ra[b], zp_hbm.at[pl.ds(0, BCH)], os_[b]).wait()

  def add(b):
    va, vb = ra[b], rb[b]

    @pl.loop(0, BCH)
    def _(r):
      for q in range(4):
        sl = pl.ds(q * 16, 16)
        va[r, sl] = va[r, sl] + vb[r, sl]

  gathers_start(0, 0)

  @pl.loop(0, (NCH_D - 1) // 2)
  def _(j):
    for b in range(2):
      i = 2 * j + b
      bn = 1 - b
      if b == 0:
        @pl.when(j > 0)
        def _():
          out_wait(bn)
      else:
        out_wait(bn)
      gathers_start(i + 1, bn)
      gathers_wait(b)
      add(b)
      out_start(i, b)

  out_wait(1)
  gathers_wait(0)
  add(0)
  out_start(NCH_D - 1, 0)
  out_wait(0)


_sc_pairs = pl.kernel(
    _sc_pairs_body,
    out_type=jax.ShapeDtypeStruct((NLPAD, C), jnp.float32),
    mesh=_mesh,
    compiler_params=_sc_params,
    scratch_types=[
        pltpu.VMEM((LPT,), jnp.int32),
        pltpu.VMEM((LPT,), jnp.int32),
        pltpu.VMEM((BCH, C), jnp.float32),
        pltpu.VMEM((BCH, C), jnp.float32),
        pltpu.VMEM((BCH, C), jnp.float32),
        pltpu.VMEM((BCH, C), jnp.float32),
        pltpu.SemaphoreType.DMA,
        pltpu.SemaphoreType.DMA,
        pltpu.SemaphoreType.DMA,
        pltpu.SemaphoreType.DMA,
        pltpu.SemaphoreType.DMA,
        pltpu.SemaphoreType.DMA,
    ])


_DOT = functools.partial(jnp.dot, precision=lax.Precision.HIGHEST)
_RB = 1000


def _tc_mm1_body(x_ref, w_ref, o_ref):
  o_ref[...] = _DOT(x_ref[...], w_ref[...])


def _tc_mm1(x, w1):
  return pl.pallas_call(
      _tc_mm1_body,
      grid=(N // _RB,),
      in_specs=[pl.BlockSpec((_RB, F_IN), lambda i: (i, 0)),
                pl.BlockSpec((F_IN, C), lambda i: (0, 0))],
      out_specs=pl.BlockSpec((_RB, C), lambda i: (i, 0)),
      out_shape=jax.ShapeDtypeStruct((N, C), jnp.float32),
  )(x, w1)


def _tc_mid_body(acc_ref, hp_ref, dis_ref, b_ref, a_ref, w_ref, o_ref):
  dis = dis_ref[...]
  acc = acc_ref[0] + acc_ref[1]
  pre = (acc + dis * hp_ref[...]) * dis + b_ref[...]
  h = jnp.where(pre >= 0, pre, a_ref[0, 0] * pre)
  o_ref[...] = _DOT(h, w_ref[...])


def _tc_mid(acc, hp, dis, b, a, w):
  return pl.pallas_call(
      _tc_mid_body,
      grid=(N // _RB,),
      in_specs=[pl.BlockSpec((NC, _RB, C), lambda i: (0, i, 0)),
                pl.BlockSpec((_RB, C), lambda i: (i, 0)),
                pl.BlockSpec((_RB, 1), lambda i: (i, 0)),
                pl.BlockSpec((1, C), lambda i: (0, 0)),
                pl.BlockSpec((1, 1), lambda i: (0, 0)),
                pl.BlockSpec((C, C), lambda i: (0, 0))],
      out_specs=pl.BlockSpec((_RB, C), lambda i: (i, 0)),
      out_shape=jax.ShapeDtypeStruct((N, C), jnp.float32),
  )(acc, hp, dis, b, a, w)


def _tc_head_body(acc_ref, hp_ref, dis_ref, b_ref, a_ref, wa_ref, wb_ref,
                  bd_ref, oa_ref, ob_ref):
  dis = dis_ref[...]
  acc = acc_ref[0] + acc_ref[1]
  pre = (acc + dis * hp_ref[...]) * dis + b_ref[...]
  h = jnp.where(pre >= 0, pre, a_ref[0, 0] * pre)
  oa_ref[...] = _DOT(h, wa_ref[...]) + bd_ref[...]
  ob_ref[...] = _DOT(h, wb_ref[...])


def _tc_head(acc, hp, dis, b, a, wa, wb, bd):
  return pl.pallas_call(
      _tc_head_body,
      grid=(N // _RB,),
      in_specs=[pl.BlockSpec((NC, _RB, C), lambda i: (0, i, 0)),
                pl.BlockSpec((_RB, C), lambda i: (i, 0)),
                pl.BlockSpec((_RB, 1), lambda i: (i, 0)),
                pl.BlockSpec((1, C), lambda i: (0, 0)),
                pl.BlockSpec((1, 1), lambda i: (0, 0)),
                pl.BlockSpec((C, C), lambda i: (0, 0)),
                pl.BlockSpec((C, C), lambda i: (0, 0)),
                pl.BlockSpec((1, C), lambda i: (0, 0))],
      out_specs=[pl.BlockSpec((_RB, C), lambda i: (i, 0)),
                 pl.BlockSpec((_RB, C), lambda i: (i, 0))],
      out_shape=[jax.ShapeDtypeStruct((N, C), jnp.float32),
                 jax.ShapeDtypeStruct((N, C), jnp.float32)],
  )(acc, hp, dis, b, a, wa, wb, bd)


_LB = 6400


def _tc_out_body(zp_ref, a_ref, w_ref, b_ref, o_ref):
  z = zp_ref[...]
  z = jnp.where(z >= 0, z, a_ref[0, 0] * z)
  o_ref[...] = _DOT(z, w_ref[...]) + b_ref[0, 0]


def _tc_out(zp, a, w, b):
  return pl.pallas_call(
      _tc_out_body,
      grid=(NLPAD // _LB,),
      in_specs=[pl.BlockSpec((_LB, C), lambda i: (i, 0)),
                pl.BlockSpec((1, 1), lambda i: (0, 0)),
                pl.BlockSpec((C, 1), lambda i: (0, 0)),
                pl.BlockSpec((1, 1), lambda i: (0, 0))],
      out_specs=pl.BlockSpec((_LB, 1), lambda i: (i, 0)),
      out_shape=jax.ShapeDtypeStruct((NLPAD, 1), jnp.float32),
  )(zp, a, w, b)


def kernel(x, edge_index, edge_weight, label_edge_index,
           W1, b1, a1, W2, b2, a2, Wd1, bd1, ad, Wd2, bd2):
  i32 = jnp.int32
  f32 = jnp.float32
  src = edge_index[0].astype(i32)
  dst = edge_index[1].astype(i32)
  ew = edge_weight.astype(f32)
  epad = EPAD - E
  src_p = jnp.concatenate([src, jnp.zeros((epad,), i32)])
  dst_p = jnp.concatenate([dst, jnp.zeros((epad,), i32)])
  ew_p = jnp.concatenate([ew, jnp.zeros((epad,), f32)])
  lpad = NLPAD - NL
  l0_p = jnp.concatenate([label_edge_index[0].astype(i32),
                          jnp.zeros((lpad,), i32)])
  l1_p = jnp.concatenate([label_edge_index[1].astype(i32),
                          jnp.zeros((lpad,), i32)])

  zdeg = jnp.zeros((DROWS, 16), f32)
  iota_rows = jnp.arange(DROWS, dtype=i32).reshape(DROWS // BCH, BCH)
  zrows = jnp.zeros((ROWS_PT, C), f32)
  dst2 = dst_p.reshape(NW, NCH_B, BCH)

  dis2d, w_e = _sc_deg(src_p, dst_p, ew_p, zdeg, iota_rows)
  dis = dis2d.reshape(NPAD)[:N].reshape(N, 1)

  h1p = _tc_mm1(x, W1)
  acc1 = _sc_scatter(h1p, src_p, dst2, w_e, zrows)
  h2p = _tc_mid(acc1, h1p, dis, b1.reshape(1, C), a1.reshape(1, 1), W2)

  acc2 = _sc_scatter(h2p, src_p, dst2, w_e, zrows)
  A, B = _tc_head(acc2, h2p, dis, b2.reshape(1, C), a2.reshape(1, 1),
                  Wd1[:C], Wd1[C:], bd1.reshape(1, C))

  zp = _sc_pairs(A, B, l0_p, l1_p)
  out = _tc_out(zp, ad.reshape(1, 1), Wd2, bd2.reshape(1, 1))
  return out[:NL]


# speedup vs baseline: 14.9306x; 1.0518x over previous
import functools

import jax
import jax.numpy as jnp
from jax import lax
from jax.experimental import pallas as pl
from jax.experimental.pallas import tpu as pltpu
from jax.experimental.pallas import tpu_sc as plsc

N = 10000
NPAD = 10240
E = 320000
EPAD = 323584
NL = 100000
NLPAD = 102400
F_IN = 128
C = 64

NC = 2
NS = 16
NW = NC * NS

EPT_A = EPAD // NS
NVEC_A = EPT_A // 16
DROWS = NPAD // 16
DROWS_PT = DROWS // NS

EPT_B = EPAD // NW
BCH = 128
NCH_B = EPT_B // BCH
ROWS_PT = NPAD // NS

LPT = NLPAD // NW
NCH_D = LPT // BCH

_mesh = plsc.VectorSubcoreMesh(
    core_axis_name="c", subcore_axis_name="s", num_cores=NC, num_subcores=NS)
_sc_params = pltpu.CompilerParams(needs_layout_passes=False,
                                  use_tc_tiling_on_sc=False)


def _rsqrt_newton(d):
  i = plsc.bitcast(d, jnp.int32)
  i = jnp.int32(0x5F3759DF) - lax.shift_right_logical(i, 1)
  z = plsc.bitcast(i, jnp.float32)
  half = d * 0.5
  for _ in range(3):
    z = z * (1.5 - half * z * z)
  return z


def _sc_deg_body(src_hbm, dst_hbm, ew_hbm, zdeg_hbm, iota_hbm,
                 dis_hbm, w_hbm,
                 src_v, dst_v, ew_v, part_v, idx_v, tmp_v, w_v, acc_ref, sem):
  c = lax.axis_index("c")
  s = lax.axis_index("s")

  @pl.when(c == 0)
  def _():
    base = s * EPT_A
    pltpu.sync_copy(dst_hbm.at[pl.ds(base, EPT_A)], dst_v)
    pltpu.sync_copy(ew_hbm.at[pl.ds(base, EPT_A)], ew_v)
    pltpu.sync_copy(zdeg_hbm, part_v)
    pltpu.sync_copy(iota_hbm, idx_v)
    pltpu.sync_copy(zdeg_hbm.at[pl.ds(s * DROWS_PT, DROWS_PT)],
                    acc_ref.at[pl.ds(s * DROWS_PT, DROWS_PT)])

    @pl.loop(0, NVEC_A, unroll=4)
    def _(i):
      d16 = dst_v[pl.ds(i * 16, 16)]
      e16 = ew_v[pl.ds(i * 16, 16)]
      plsc.addupdate_scatter(
          part_v,
          [lax.shift_right_logical(d16, 4), jnp.bitwise_and(d16, 15)], e16)

    plsc.subcore_barrier()
    @pl.loop(0, DROWS // BCH)
    def _(j):
      pltpu.async_copy(part_v.at[pl.ds(j * BCH, BCH)],
                       acc_ref.at[idx_v.at[j]], sem, add=True).wait()
    plsc.subcore_barrier()

    rbase = s * DROWS_PT
    pltpu.sync_copy(acc_ref.at[pl.ds(rbase, DROWS_PT)], tmp_v)

    @pl.loop(0, DROWS_PT)
    def _(r):
      tmp_v[r] = _rsqrt_newton(tmp_v[r] + 1.0)

    pltpu.sync_copy(tmp_v, acc_ref.at[pl.ds(rbase, DROWS_PT)])
    pltpu.sync_copy(tmp_v, dis_hbm.at[pl.ds(rbase, DROWS_PT)])
    plsc.subcore_barrier()
    pltpu.sync_copy(acc_ref, part_v)

    pltpu.sync_copy(src_hbm.at[pl.ds(base, EPT_A)], src_v)

    @plsc.parallel_loop(0, NVEC_A, unroll=4)
    def _(i):
      s16 = src_v[pl.ds(i * 16, 16)]
      d16 = plsc.load_gather(
          part_v,
          [lax.shift_right_logical(s16, 4), jnp.bitwise_and(s16, 15)])
      w_v[pl.ds(i * 16, 16)] = d16 * ew_v[pl.ds(i * 16, 16)]

    pltpu.sync_copy(w_v, w_hbm.at[pl.ds(base, EPT_A)])


_sc_deg = pl.kernel(
    _sc_deg_body,
    out_type=[jax.ShapeDtypeStruct((DROWS, 16), jnp.float32),
              jax.ShapeDtypeStruct((EPAD,), jnp.float32)],
    mesh=_mesh,
    compiler_params=_sc_params,
    scratch_types=[
        pltpu.VMEM((EPT_A,), jnp.int32),
        pltpu.VMEM((EPT_A,), jnp.int32),
        pltpu.VMEM((EPT_A,), jnp.float32),
        pltpu.VMEM((DROWS, 16), jnp.float32),
        pltpu.VMEM((DROWS // BCH, BCH), jnp.int32),
        pltpu.VMEM((DROWS_PT, 16), jnp.float32),
        pltpu.VMEM((EPT_A,), jnp.float32),
        pltpu.VMEM_SHARED((DROWS, 16), jnp.float32),
        pltpu.SemaphoreType.DMA,
    ])


def _sc_scatter_body(h_hbm, src_hbm, dst2_hbm, w_hbm, zrows_hbm,
                     out_hbm,
                     src_v, dst_v, w_v, r0, r1, r2, r3,
                     acc_ref, g0, g1, g2, g3, s0, s1, s2, s3):
  c = lax.axis_index("c")
  s = lax.axis_index("s")
  wid = c * NS + s
  base = wid * EPT_B
  rows = [r0, r1, r2, r3]
  gs = [g0, g1, g2, g3]
  ss = [s0, s1, s2, s3]

  pltpu.sync_copy(src_hbm.at[pl.ds(base, EPT_B)], src_v)
  pltpu.sync_copy(dst2_hbm.at[wid], dst_v)
  pltpu.sync_copy(w_hbm.at[pl.ds(base, EPT_B)], w_v)
  pltpu.sync_copy(zrows_hbm, acc_ref.at[pl.ds(s * ROWS_PT, ROWS_PT)])
  plsc.subcore_barrier()

  def gather_start(ch, b):
    pltpu.async_copy(h_hbm.at[src_v.at[pl.ds(ch * BCH, BCH)]], rows[b], gs[b])

  def gather_wait(b):
    pltpu.make_async_copy(h_hbm.at[pl.ds(0, BCH)], rows[b], gs[b]).wait()

  def scatter_start(ch, b):
    pltpu.async_copy(rows[b], acc_ref.at[dst_v.at[ch]], ss[b], add=True)

  def scatter_wait(b):
    pltpu.make_async_copy(h_hbm.at[pl.ds(0, BCH)], rows[b], ss[b]).wait()

  def scale(ch, b):
    rb = rows[b]

    @plsc.parallel_loop(0, BCH, unroll=8)
    def _(e):
      wv = plsc.load_gather(w_v, [jnp.full((16,), ch * BCH + e, jnp.int32)])
      for q in range(4):
        sl = pl.ds(q * 16, 16)
        rb[e, sl] = rb[e, sl] * wv

  gather_start(0, 0)

  @pl.loop(0, (NCH_B - 3) // 4)
  def _(j):
    for b in range(4):
      i = 4 * j + b
      bn = (b + 1) % 4
      if b < 3:
        @pl.when(j > 0)
        def _():
          scatter_wait(bn)
      else:
        scatter_wait(bn)
      gather_start(i + 1, bn)
      gather_wait(b)
      scale(i, b)
      scatter_start(i, b)

  for i in (NCH_B - 3, NCH_B - 2, NCH_B - 1):
    b = i % 4
    bn = (b + 1) % 4
    if i + 1 < NCH_B:
      scatter_wait(bn)
      gather_start(i + 1, bn)
    gather_wait(b)
    scale(i, b)
    scatter_start(i, b)
  scatter_wait(3)
  scatter_wait(0)
  scatter_wait(1)
  scatter_wait(2)

  plsc.subcore_barrier()
  pltpu.sync_copy(acc_ref.at[pl.ds(s * ROWS_PT, ROWS_PT)],
                  out_hbm.at[c, pl.ds(s * ROWS_PT, ROWS_PT)])


_sc_scatter = pl.kernel(
    _sc_scatter_body,
    out_type=jax.ShapeDtypeStruct((NC, NPAD, C), jnp.float32),
    mesh=_mesh,
    compiler_params=_sc_params,
    scratch_types=[
        pltpu.VMEM((EPT_B,), jnp.int32),
        pltpu.VMEM((NCH_B, BCH), jnp.int32),
        pltpu.VMEM((EPT_B,), jnp.float32),
        pltpu.VMEM((BCH, C), jnp.float32),
        pltpu.VMEM((BCH, C), jnp.float32),
        pltpu.VMEM((BCH, C), jnp.float32),
        pltpu.VMEM((BCH, C), jnp.float32),
        pltpu.VMEM_SHARED((NPAD, C), jnp.float32),
        pltpu.SemaphoreType.DMA,
        pltpu.SemaphoreType.DMA,
        pltpu.SemaphoreType.DMA,
        pltpu.SemaphoreType.DMA,
        pltpu.SemaphoreType.DMA,
        pltpu.SemaphoreType.DMA,
        pltpu.SemaphoreType.DMA,
        pltpu.SemaphoreType.DMA,
    ])


def _sc_pairs_body(a_hbm, b_hbm, l0_hbm, l1_hbm,
                   zp_hbm,
                   l0_v, l1_v, ra0, ra1, rb0, rb1,
                   ga0, ga1, gb0, gb1, os0, os1):
  c = lax.axis_index("c")
  s = lax.axis_index("s")
  wid = c * NS + s
  base = wid * LPT
  ra = [ra0, ra1]
  rb = [rb0, rb1]
  ga = [ga0, ga1]
  gb = [gb0, gb1]
  os_ = [os0, os1]
  pltpu.sync_copy(l0_hbm.at[pl.ds(base, LPT)], l0_v)
  pltpu.sync_copy(l1_hbm.at[pl.ds(base, LPT)], l1_v)

  def gathers_start(ch, b):
    pltpu.async_copy(a_hbm.at[l0_v.at[pl.ds(ch * BCH, BCH)]], ra[b], ga[b])
    pltpu.async_copy(b_hbm.at[l1_v.at[pl.ds(ch * BCH, BCH)]], rb[b], gb[b])

  def gathers_wait(b):
    pltpu.make_async_copy(a_hbm.at[pl.ds(0, BCH)], ra[b], ga[b]).wait()
    pltpu.make_async_copy(b_hbm.at[pl.ds(0, BCH)], rb[b], gb[b]).wait()

  def out_start(ch, b):
    pltpu.async_copy(ra[b], zp_hbm.at[pl.ds(base + ch * BCH, BCH)], os_[b])

  def out_wait(b):
    pltpu.make_async_copy(ra[b], zp_hbm.at[pl.ds(0, BCH)], os_[b]).wait()

  def add(b):
    va, vb = ra[b], rb[b]

    @plsc.parallel_loop(0, BCH, unroll=8)
    def _(r):
      for q in range(4):
        sl = pl.ds(q * 16, 16)
        va[r, sl] = va[r, sl] + vb[r, sl]

  gathers_start(0, 0)

  @pl.loop(0, (NCH_D - 1) // 2)
  def _(j):
    for b in range(2):
      i = 2 * j + b
      bn = 1 - b
      if b == 0:
        @pl.when(j > 0)
        def _():
          out_wait(bn)
      else:
        out_wait(bn)
      gathers_start(i + 1, bn)
      gathers_wait(b)
      add(b)
      out_start(i, b)

  out_wait(1)
  gathers_wait(0)
  add(0)
  out_start(NCH_D - 1, 0)
  out_wait(0)


_sc_pairs = pl.kernel(
    _sc_pairs_body,
    out_type=jax.ShapeDtypeStruct((NLPAD, C), jnp.float32),
    mesh=_mesh,
    compiler_params=_sc_params,
    scratch_types=[
        pltpu.VMEM((LPT,), jnp.int32),
        pltpu.VMEM((LPT,), jnp.int32),
        pltpu.VMEM((BCH, C), jnp.float32),
        pltpu.VMEM((BCH, C), jnp.float32),
        pltpu.VMEM((BCH, C), jnp.float32),
        pltpu.VMEM((BCH, C), jnp.float32),
        pltpu.SemaphoreType.DMA,
        pltpu.SemaphoreType.DMA,
        pltpu.SemaphoreType.DMA,
        pltpu.SemaphoreType.DMA,
        pltpu.SemaphoreType.DMA,
        pltpu.SemaphoreType.DMA,
    ])


_DOT = functools.partial(jnp.dot, precision=lax.Precision.HIGHEST)
_RB = 1000


def _tc_mm1_body(x_ref, w_ref, o_ref):
  o_ref[...] = _DOT(x_ref[...], w_ref[...])


def _tc_mm1(x, w1):
  return pl.pallas_call(
      _tc_mm1_body,
      grid=(N // _RB,),
      in_specs=[pl.BlockSpec((_RB, F_IN), lambda i: (i, 0)),
                pl.BlockSpec((F_IN, C), lambda i: (0, 0))],
      out_specs=pl.BlockSpec((_RB, C), lambda i: (i, 0)),
      out_shape=jax.ShapeDtypeStruct((N, C), jnp.float32),
  )(x, w1)


def _tc_mid_body(acc_ref, hp_ref, dis_ref, b_ref, a_ref, w_ref, o_ref):
  dis = dis_ref[...]
  acc = acc_ref[0] + acc_ref[1]
  pre = (acc + dis * hp_ref[...]) * dis + b_ref[...]
  h = jnp.where(pre >= 0, pre, a_ref[0, 0] * pre)
  o_ref[...] = _DOT(h, w_ref[...])


def _tc_mid(acc, hp, dis, b, a, w):
  return pl.pallas_call(
      _tc_mid_body,
      grid=(N // _RB,),
      in_specs=[pl.BlockSpec((NC, _RB, C), lambda i: (0, i, 0)),
                pl.BlockSpec((_RB, C), lambda i: (i, 0)),
                pl.BlockSpec((_RB, 1), lambda i: (i, 0)),
                pl.BlockSpec((1, C), lambda i: (0, 0)),
                pl.BlockSpec((1, 1), lambda i: (0, 0)),
                pl.BlockSpec((C, C), lambda i: (0, 0))],
      out_specs=pl.BlockSpec((_RB, C), lambda i: (i, 0)),
      out_shape=jax.ShapeDtypeStruct((N, C), jnp.float32),
  )(acc, hp, dis, b, a, w)


def _tc_head_body(acc_ref, hp_ref, dis_ref, b_ref, a_ref, wa_ref, wb_ref,
                  bd_ref, oa_ref, ob_ref):
  dis = dis_ref[...]
  acc = acc_ref[0] + acc_ref[1]
  pre = (acc + dis * hp_ref[...]) * dis + b_ref[...]
  h = jnp.where(pre >= 0, pre, a_ref[0, 0] * pre)
  oa_ref[...] = _DOT(h, wa_ref[...]) + bd_ref[...]
  ob_ref[...] = _DOT(h, wb_ref[...])


def _tc_head(acc, hp, dis, b, a, wa, wb, bd):
  return pl.pallas_call(
      _tc_head_body,
      grid=(N // _RB,),
      in_specs=[pl.BlockSpec((NC, _RB, C), lambda i: (0, i, 0)),
                pl.BlockSpec((_RB, C), lambda i: (i, 0)),
                pl.BlockSpec((_RB, 1), lambda i: (i, 0)),
                pl.BlockSpec((1, C), lambda i: (0, 0)),
                pl.BlockSpec((1, 1), lambda i: (0, 0)),
                pl.BlockSpec((C, C), lambda i: (0, 0)),
                pl.BlockSpec((C, C), lambda i: (0, 0)),
                pl.BlockSpec((1, C), lambda i: (0, 0))],
      out_specs=[pl.BlockSpec((_RB, C), lambda i: (i, 0)),
                 pl.BlockSpec((_RB, C), lambda i: (i, 0))],
      out_shape=[jax.ShapeDtypeStruct((N, C), jnp.float32),
                 jax.ShapeDtypeStruct((N, C), jnp.float32)],
  )(acc, hp, dis, b, a, wa, wb, bd)


_LB = 6400


def _tc_out_body(zp_ref, a_ref, w_ref, b_ref, o_ref):
  z = zp_ref[...]
  z = jnp.where(z >= 0, z, a_ref[0, 0] * z)
  o_ref[...] = _DOT(z, w_ref[...]) + b_ref[0, 0]


def _tc_out(zp, a, w, b):
  return pl.pallas_call(
      _tc_out_body,
      grid=(NLPAD // _LB,),
      in_specs=[pl.BlockSpec((_LB, C), lambda i: (i, 0)),
                pl.BlockSpec((1, 1), lambda i: (0, 0)),
                pl.BlockSpec((C, 1), lambda i: (0, 0)),
                pl.BlockSpec((1, 1), lambda i: (0, 0))],
      out_specs=pl.BlockSpec((_LB, 1), lambda i: (i, 0)),
      out_shape=jax.ShapeDtypeStruct((NLPAD, 1), jnp.float32),
  )(zp, a, w, b)


def kernel(x, edge_index, edge_weight, label_edge_index,
           W1, b1, a1, W2, b2, a2, Wd1, bd1, ad, Wd2, bd2):
  i32 = jnp.int32
  f32 = jnp.float32
  src = edge_index[0].astype(i32)
  dst = edge_index[1].astype(i32)
  ew = edge_weight.astype(f32)
  epad = EPAD - E
  src_p = jnp.concatenate([src, jnp.zeros((epad,), i32)])
  dst_p = jnp.concatenate([dst, jnp.zeros((epad,), i32)])
  ew_p = jnp.concatenate([ew, jnp.zeros((epad,), f32)])
  lpad = NLPAD - NL
  l0_p = jnp.concatenate([label_edge_index[0].astype(i32),
                          jnp.zeros((lpad,), i32)])
  l1_p = jnp.concatenate([label_edge_index[1].astype(i32),
                          jnp.zeros((lpad,), i32)])

  zdeg = jnp.zeros((DROWS, 16), f32)
  iota_rows = jnp.arange(DROWS, dtype=i32).reshape(DROWS // BCH, BCH)
  zrows = jnp.zeros((ROWS_PT, C), f32)
  dst2 = dst_p.reshape(NW, NCH_B, BCH)

  dis2d, w_e = _sc_deg(src_p, dst_p, ew_p, zdeg, iota_rows)
  dis = dis2d.reshape(NPAD)[:N].reshape(N, 1)

  h1p = _tc_mm1(x, W1)
  acc1 = _sc_scatter(h1p, src_p, dst2, w_e, zrows)
  h2p = _tc_mid(acc1, h1p, dis, b1.reshape(1, C), a1.reshape(1, 1), W2)

  acc2 = _sc_scatter(h2p, src_p, dst2, w_e, zrows)
  A, B = _tc_head(acc2, h2p, dis, b2.reshape(1, C), a2.reshape(1, 1),
                  Wd1[:C], Wd1[C:], bd1.reshape(1, C))

  zp = _sc_pairs(A, B, l0_p, l1_p)
  out = _tc_out(zp, ad.reshape(1, 1), Wd2, bd2.reshape(1, 1))
  return out[:NL]


# speedup vs baseline: 15.1246x; 1.0130x over previous
import functools

import jax
import jax.numpy as jnp
from jax import lax
from jax.experimental import pallas as pl
from jax.experimental.pallas import tpu as pltpu
from jax.experimental.pallas import tpu_sc as plsc

N = 10000
NPAD = 10240
E = 320000
EPAD = 323584
NL = 100000
NLPAD = 102400
F_IN = 128
C = 64

NC = 2
NS = 16
NW = NC * NS

EPT_A = EPAD // NS
NVEC_A = EPT_A // 16
DROWS = NPAD // 16
DROWS_PT = DROWS // NS

EPT_B = EPAD // NW
BCH = 128
NCH_B = EPT_B // BCH
ROWS_PT = NPAD // NS

LPT = NLPAD // NW
NCH_D = LPT // BCH

_mesh = plsc.VectorSubcoreMesh(
    core_axis_name="c", subcore_axis_name="s", num_cores=NC, num_subcores=NS)
_sc_params = pltpu.CompilerParams(needs_layout_passes=False,
                                  use_tc_tiling_on_sc=False)


def _rsqrt_newton(d):
  i = plsc.bitcast(d, jnp.int32)
  i = jnp.int32(0x5F3759DF) - lax.shift_right_logical(i, 1)
  z = plsc.bitcast(i, jnp.float32)
  half = d * 0.5
  for _ in range(3):
    z = z * (1.5 - half * z * z)
  return z


def _sc_deg_body(src_hbm, dst_hbm, ew_hbm, zdeg_hbm, iota_hbm,
                 dis_hbm, w_hbm,
                 src_v, dst_v, ew_v, part_v, idx_v, tmp_v, w_v, acc_ref, sem):
  c = lax.axis_index("c")
  s = lax.axis_index("s")

  @pl.when(c == 0)
  def _():
    base = s * EPT_A
    pltpu.sync_copy(dst_hbm.at[pl.ds(base, EPT_A)], dst_v)
    pltpu.sync_copy(ew_hbm.at[pl.ds(base, EPT_A)], ew_v)
    pltpu.sync_copy(zdeg_hbm, part_v)
    pltpu.sync_copy(iota_hbm, idx_v)
    pltpu.sync_copy(zdeg_hbm.at[pl.ds(s * DROWS_PT, DROWS_PT)],
                    acc_ref.at[pl.ds(s * DROWS_PT, DROWS_PT)])

    @pl.loop(0, NVEC_A, unroll=4)
    def _(i):
      d16 = dst_v[pl.ds(i * 16, 16)]
      e16 = ew_v[pl.ds(i * 16, 16)]
      plsc.addupdate_scatter(
          part_v,
          [lax.shift_right_logical(d16, 4), jnp.bitwise_and(d16, 15)], e16)

    plsc.subcore_barrier()
    @pl.loop(0, DROWS // BCH)
    def _(j):
      pltpu.async_copy(part_v.at[pl.ds(j * BCH, BCH)],
                       acc_ref.at[idx_v.at[j]], sem, add=True).wait()
    plsc.subcore_barrier()

    rbase = s * DROWS_PT
    pltpu.sync_copy(acc_ref.at[pl.ds(rbase, DROWS_PT)], tmp_v)

    @pl.loop(0, DROWS_PT)
    def _(r):
      tmp_v[r] = _rsqrt_newton(tmp_v[r] + 1.0)

    pltpu.sync_copy(tmp_v, acc_ref.at[pl.ds(rbase, DROWS_PT)])
    pltpu.sync_copy(tmp_v, dis_hbm.at[pl.ds(rbase, DROWS_PT)])
    plsc.subcore_barrier()
    pltpu.sync_copy(acc_ref, part_v)

    pltpu.sync_copy(src_hbm.at[pl.ds(base, EPT_A)], src_v)

    @plsc.parallel_loop(0, NVEC_A, unroll=4)
    def _(i):
      s16 = src_v[pl.ds(i * 16, 16)]
      d16 = plsc.load_gather(
          part_v,
          [lax.shift_right_logical(s16, 4), jnp.bitwise_and(s16, 15)])
      w_v[pl.ds(i * 16, 16)] = d16 * ew_v[pl.ds(i * 16, 16)]

    pltpu.sync_copy(w_v, w_hbm.at[pl.ds(base, EPT_A)])


_sc_deg = pl.kernel(
    _sc_deg_body,
    out_type=[jax.ShapeDtypeStruct((DROWS, 16), jnp.float32),
              jax.ShapeDtypeStruct((EPAD,), jnp.float32)],
    mesh=_mesh,
    compiler_params=_sc_params,
    scratch_types=[
        pltpu.VMEM((EPT_A,), jnp.int32),
        pltpu.VMEM((EPT_A,), jnp.int32),
        pltpu.VMEM((EPT_A,), jnp.float32),
        pltpu.VMEM((DROWS, 16), jnp.float32),
        pltpu.VMEM((DROWS // BCH, BCH), jnp.int32),
        pltpu.VMEM((DROWS_PT, 16), jnp.float32),
        pltpu.VMEM((EPT_A,), jnp.float32),
        pltpu.VMEM_SHARED((DROWS, 16), jnp.float32),
        pltpu.SemaphoreType.DMA,
    ])


def _sc_scatter_body(h_hbm, src_hbm, dst2_hbm, w_hbm, zrows_hbm,
                     out_hbm,
                     src_v, dst_v, w_v,
                     r0, r1, r2, r3, r4, r5,
                     acc_ref,
                     g0, g1, g2, g3, g4, g5,
                     s0, s1, s2, s3, s4, s5):
  c = lax.axis_index("c")
  s = lax.axis_index("s")
  wid = c * NS + s
  base = wid * EPT_B
  rows = [r0, r1, r2, r3, r4, r5]
  gs = [g0, g1, g2, g3, g4, g5]
  ss = [s0, s1, s2, s3, s4, s5]

  pltpu.sync_copy(src_hbm.at[pl.ds(base, EPT_B)], src_v)
  pltpu.sync_copy(dst2_hbm.at[wid], dst_v)
  pltpu.sync_copy(w_hbm.at[pl.ds(base, EPT_B)], w_v)
  pltpu.sync_copy(zrows_hbm, acc_ref.at[pl.ds(s * ROWS_PT, ROWS_PT)])
  plsc.subcore_barrier()

  def gather_start(ch, b):
    pltpu.async_copy(h_hbm.at[src_v.at[pl.ds(ch * BCH, BCH)]], rows[b], gs[b])

  def gather_wait(b):
    pltpu.make_async_copy(h_hbm.at[pl.ds(0, BCH)], rows[b], gs[b]).wait()

  def scatter_start(ch, b):
    pltpu.async_copy(rows[b], acc_ref.at[dst_v.at[ch]], ss[b], add=True)

  def scatter_wait(b):
    pltpu.make_async_copy(h_hbm.at[pl.ds(0, BCH)], rows[b], ss[b]).wait()

  def scale(ch, b):
    rb = rows[b]

    @plsc.parallel_loop(0, BCH, unroll=8)
    def _(e):
      wv = plsc.load_gather(w_v, [jnp.full((16,), ch * BCH + e, jnp.int32)])
      for q in range(4):
        sl = pl.ds(q * 16, 16)
        rb[e, sl] = rb[e, sl] * wv

  NB, LA = 6, 3
  NMAIN = (NCH_B // NB) * NB - NB
  assert NMAIN % NB == 0 and NMAIN - 1 + LA < NCH_B
  for k in range(LA):
    gather_start(k, k)

  @pl.loop(0, NMAIN // NB)
  def _(j):
    for b in range(NB):
      i = NB * j + b
      bn = (b + LA) % NB
      if b < LA:
        @pl.when(j > 0)
        def _():
          scatter_wait(bn)
      else:
        scatter_wait(bn)
      gather_start(i + LA, bn)
      gather_wait(b)
      scale(i, b)
      scatter_start(i, b)

  for i in range(NMAIN, NCH_B):
    b = i % NB
    bn = (b + LA) % NB
    if i + LA < NCH_B:
      scatter_wait(bn)
      gather_start(i + LA, bn)
    gather_wait(b)
    scale(i, b)
    scatter_start(i, b)
  for i in range(NCH_B - NB, NCH_B):
    scatter_wait(i % NB)

  plsc.subcore_barrier()
  pltpu.sync_copy(acc_ref.at[pl.ds(s * ROWS_PT, ROWS_PT)],
                  out_hbm.at[c, pl.ds(s * ROWS_PT, ROWS_PT)])


_sc_scatter = pl.kernel(
    _sc_scatter_body,
    out_type=jax.ShapeDtypeStruct((NC, NPAD, C), jnp.float32),
    mesh=_mesh,
    compiler_params=_sc_params,
    scratch_types=[
        pltpu.VMEM((EPT_B,), jnp.int32),
        pltpu.VMEM((NCH_B, BCH), jnp.int32),
        pltpu.VMEM((EPT_B,), jnp.float32),
    ] + [pltpu.VMEM((BCH, C), jnp.float32)] * 6
    + [pltpu.VMEM_SHARED((NPAD, C), jnp.float32)]
    + [pltpu.SemaphoreType.DMA] * 12)


def _sc_pairs_body(a_hbm, b_hbm, l0_hbm, l1_hbm,
                   zp_hbm,
                   l0_v, l1_v, ra0, ra1, ra2, ra3, rb0, rb1, rb2, rb3,
                   ga0, ga1, ga2, ga3, gb0, gb1, gb2, gb3,
                   os0, os1, os2, os3):
  c = lax.axis_index("c")
  s = lax.axis_index("s")
  wid = c * NS + s
  base = wid * LPT
  ra = [ra0, ra1, ra2, ra3]
  rb = [rb0, rb1, rb2, rb3]
  ga = [ga0, ga1, ga2, ga3]
  gb = [gb0, gb1, gb2, gb3]
  os_ = [os0, os1, os2, os3]
  pltpu.sync_copy(l0_hbm.at[pl.ds(base, LPT)], l0_v)
  pltpu.sync_copy(l1_hbm.at[pl.ds(base, LPT)], l1_v)

  def gathers_start(ch, b):
    pltpu.async_copy(a_hbm.at[l0_v.at[pl.ds(ch * BCH, BCH)]], ra[b], ga[b])
    pltpu.async_copy(b_hbm.at[l1_v.at[pl.ds(ch * BCH, BCH)]], rb[b], gb[b])

  def gathers_wait(b):
    pltpu.make_async_copy(a_hbm.at[pl.ds(0, BCH)], ra[b], ga[b]).wait()
    pltpu.make_async_copy(b_hbm.at[pl.ds(0, BCH)], rb[b], gb[b]).wait()

  def out_start(ch, b):
    pltpu.async_copy(ra[b], zp_hbm.at[pl.ds(base + ch * BCH, BCH)], os_[b])

  def out_wait(b):
    pltpu.make_async_copy(ra[b], zp_hbm.at[pl.ds(0, BCH)], os_[b]).wait()

  def add(b):
    va, vb = ra[b], rb[b]

    @plsc.parallel_loop(0, BCH, unroll=8)
    def _(r):
      for q in range(4):
        sl = pl.ds(q * 16, 16)
        va[r, sl] = va[r, sl] + vb[r, sl]

  NB, LA = 4, 2
  NMAIN = ((NCH_D - LA) // NB) * NB
  for k in range(LA):
    gathers_start(k, k)

  @pl.loop(0, NMAIN // NB)
  def _(j):
    for b in range(NB):
      i = NB * j + b
      bn = (b + LA) % NB
      if b < LA:
        @pl.when(j > 0)
        def _():
          out_wait(bn)
      else:
        out_wait(bn)
      gathers_start(i + LA, bn)
      gathers_wait(b)
      add(b)
      out_start(i, b)

  for i in range(NMAIN, NCH_D):
    b = i % NB
    bn = (b + LA) % NB
    if i + LA < NCH_D:
      out_wait(bn)
      gathers_start(i + LA, bn)
    gathers_wait(b)
    add(b)
    out_start(i, b)
  for i in range(NCH_D - NB, NCH_D):
    out_wait(i % NB)


_sc_pairs = pl.kernel(
    _sc_pairs_body,
    out_type=jax.ShapeDtypeStruct((NLPAD, C), jnp.float32),
    mesh=_mesh,
    compiler_params=_sc_params,
    scratch_types=[
        pltpu.VMEM((LPT,), jnp.int32),
        pltpu.VMEM((LPT,), jnp.int32),
    ] + [pltpu.VMEM((BCH, C), jnp.float32)] * 8
    + [pltpu.SemaphoreType.DMA] * 12)


_DOT = functools.partial(jnp.dot, precision=lax.Precision.HIGHEST)
_RB = 1000


def _tc_mm1_body(x_ref, w_ref, o_ref):
  o_ref[...] = _DOT(x_ref[...], w_ref[...])


def _tc_mm1(x, w1):
  return pl.pallas_call(
      _tc_mm1_body,
      grid=(N // _RB,),
      in_specs=[pl.BlockSpec((_RB, F_IN), lambda i: (i, 0)),
                pl.BlockSpec((F_IN, C), lambda i: (0, 0))],
      out_specs=pl.BlockSpec((_RB, C), lambda i: (i, 0)),
      out_shape=jax.ShapeDtypeStruct((N, C), jnp.float32),
  )(x, w1)


def _tc_mid_body(acc_ref, hp_ref, dis_ref, b_ref, a_ref, w_ref, o_ref):
  dis = dis_ref[...]
  acc = acc_ref[0] + acc_ref[1]
  pre = (acc + dis * hp_ref[...]) * dis + b_ref[...]
  h = jnp.where(pre >= 0, pre, a_ref[0, 0] * pre)
  o_ref[...] = _DOT(h, w_ref[...])


def _tc_mid(acc, hp, dis, b, a, w):
  return pl.pallas_call(
      _tc_mid_body,
      grid=(N // _RB,),
      in_specs=[pl.BlockSpec((NC, _RB, C), lambda i: (0, i, 0)),
                pl.BlockSpec((_RB, C), lambda i: (i, 0)),
                pl.BlockSpec((_RB, 1), lambda i: (i, 0)),
                pl.BlockSpec((1, C), lambda i: (0, 0)),
                pl.BlockSpec((1, 1), lambda i: (0, 0)),
                pl.BlockSpec((C, C), lambda i: (0, 0))],
      out_specs=pl.BlockSpec((_RB, C), lambda i: (i, 0)),
      out_shape=jax.ShapeDtypeStruct((N, C), jnp.float32),
  )(acc, hp, dis, b, a, w)


def _tc_head_body(acc_ref, hp_ref, dis_ref, b_ref, a_ref, wa_ref, wb_ref,
                  bd_ref, oa_ref, ob_ref):
  dis = dis_ref[...]
  acc = acc_ref[0] + acc_ref[1]
  pre = (acc + dis * hp_ref[...]) * dis + b_ref[...]
  h = jnp.where(pre >= 0, pre, a_ref[0, 0] * pre)
  oa_ref[...] = _DOT(h, wa_ref[...]) + bd_ref[...]
  ob_ref[...] = _DOT(h, wb_ref[...])


def _tc_head(acc, hp, dis, b, a, wa, wb, bd):
  return pl.pallas_call(
      _tc_head_body,
      grid=(N // _RB,),
      in_specs=[pl.BlockSpec((NC, _RB, C), lambda i: (0, i, 0)),
                pl.BlockSpec((_RB, C), lambda i: (i, 0)),
                pl.BlockSpec((_RB, 1), lambda i: (i, 0)),
                pl.BlockSpec((1, C), lambda i: (0, 0)),
                pl.BlockSpec((1, 1), lambda i: (0, 0)),
                pl.BlockSpec((C, C), lambda i: (0, 0)),
                pl.BlockSpec((C, C), lambda i: (0, 0)),
                pl.BlockSpec((1, C), lambda i: (0, 0))],
      out_specs=[pl.BlockSpec((_RB, C), lambda i: (i, 0)),
                 pl.BlockSpec((_RB, C), lambda i: (i, 0))],
      out_shape=[jax.ShapeDtypeStruct((N, C), jnp.float32),
                 jax.ShapeDtypeStruct((N, C), jnp.float32)],
  )(acc, hp, dis, b, a, wa, wb, bd)


_LB = 6400


def _tc_out_body(zp_ref, a_ref, w_ref, b_ref, o_ref):
  z = zp_ref[...]
  z = jnp.where(z >= 0, z, a_ref[0, 0] * z)
  o_ref[...] = _DOT(z, w_ref[...]) + b_ref[0, 0]


def _tc_out(zp, a, w, b):
  return pl.pallas_call(
      _tc_out_body,
      grid=(NLPAD // _LB,),
      in_specs=[pl.BlockSpec((_LB, C), lambda i: (i, 0)),
                pl.BlockSpec((1, 1), lambda i: (0, 0)),
                pl.BlockSpec((C, 1), lambda i: (0, 0)),
                pl.BlockSpec((1, 1), lambda i: (0, 0))],
      out_specs=pl.BlockSpec((_LB, 1), lambda i: (i, 0)),
      out_shape=jax.ShapeDtypeStruct((NLPAD, 1), jnp.float32),
  )(zp, a, w, b)


def kernel(x, edge_index, edge_weight, label_edge_index,
           W1, b1, a1, W2, b2, a2, Wd1, bd1, ad, Wd2, bd2):
  i32 = jnp.int32
  f32 = jnp.float32
  src = edge_index[0].astype(i32)
  dst = edge_index[1].astype(i32)
  ew = edge_weight.astype(f32)
  epad = EPAD - E
  src_p = jnp.concatenate([src, jnp.zeros((epad,), i32)])
  dst_p = jnp.concatenate([dst, jnp.zeros((epad,), i32)])
  ew_p = jnp.concatenate([ew, jnp.zeros((epad,), f32)])
  lpad = NLPAD - NL
  l0_p = jnp.concatenate([label_edge_index[0].astype(i32),
                          jnp.zeros((lpad,), i32)])
  l1_p = jnp.concatenate([label_edge_index[1].astype(i32),
                          jnp.zeros((lpad,), i32)])

  zdeg = jnp.zeros((DROWS, 16), f32)
  iota_rows = jnp.arange(DROWS, dtype=i32).reshape(DROWS // BCH, BCH)
  zrows = jnp.zeros((ROWS_PT, C), f32)
  dst2 = dst_p.reshape(NW, NCH_B, BCH)

  dis2d, w_e = _sc_deg(src_p, dst_p, ew_p, zdeg, iota_rows)
  dis = dis2d.reshape(NPAD)[:N].reshape(N, 1)

  h1p = _tc_mm1(x, W1)
  acc1 = _sc_scatter(h1p, src_p, dst2, w_e, zrows)
  h2p = _tc_mid(acc1, h1p, dis, b1.reshape(1, C), a1.reshape(1, 1), W2)

  acc2 = _sc_scatter(h2p, src_p, dst2, w_e, zrows)
  A, B = _tc_head(acc2, h2p, dis, b2.reshape(1, C), a2.reshape(1, 1),
                  Wd1[:C], Wd1[C:], bd1.reshape(1, C))

  zp = _sc_pairs(A, B, l0_p, l1_p)
  out = _tc_out(zp, ad.reshape(1, 1), Wd2, bd2.reshape(1, 1))
  return out[:NL]


# speedup vs baseline: 15.7313x; 1.0401x over previous
import functools

import jax
import jax.numpy as jnp
from jax import lax
from jax.experimental import pallas as pl
from jax.experimental.pallas import tpu as pltpu
from jax.experimental.pallas import tpu_sc as plsc

N = 10000
NPAD = 10240
E = 320000
EPAD = 323584
NL = 100000
NLPAD = 102400
F_IN = 128
C = 64

NC = 2
NS = 16
NW = NC * NS

EPT_A = EPAD // NS
NVEC_A = EPT_A // 16
DROWS = NPAD // 16
DROWS_PT = DROWS // NS

EPT_B = EPAD // NW
BCH = 128
NCH_B = EPT_B // BCH
ROWS_PT = NPAD // NS

LPT = NLPAD // NW
NCH_D = LPT // BCH

_mesh = plsc.VectorSubcoreMesh(
    core_axis_name="c", subcore_axis_name="s", num_cores=NC, num_subcores=NS)
_sc_params = pltpu.CompilerParams(needs_layout_passes=False,
                                  use_tc_tiling_on_sc=False)


def _rsqrt_newton(d):
  i = plsc.bitcast(d, jnp.int32)
  i = jnp.int32(0x5F3759DF) - lax.shift_right_logical(i, 1)
  z = plsc.bitcast(i, jnp.float32)
  half = d * 0.5
  for _ in range(3):
    z = z * (1.5 - half * z * z)
  return z


def _sc_deg_body(src_hbm, dst_hbm, ew_hbm, zdeg_hbm, iota_hbm,
                 dis_hbm, w_hbm,
                 src_v, dst_v, ew_v, part_v, idx_v, tmp_v, w_v, acc_ref, sem):
  c = lax.axis_index("c")
  s = lax.axis_index("s")

  @pl.when(c == 0)
  def _():
    base = s * EPT_A
    pltpu.sync_copy(dst_hbm.at[pl.ds(base, EPT_A)], dst_v)
    pltpu.sync_copy(ew_hbm.at[pl.ds(base, EPT_A)], ew_v)
    pltpu.sync_copy(zdeg_hbm, part_v)
    pltpu.sync_copy(iota_hbm, idx_v)
    pltpu.sync_copy(zdeg_hbm.at[pl.ds(s * DROWS_PT, DROWS_PT)],
                    acc_ref.at[pl.ds(s * DROWS_PT, DROWS_PT)])

    @pl.loop(0, NVEC_A, unroll=4)
    def _(i):
      d16 = dst_v[pl.ds(i * 16, 16)]
      e16 = ew_v[pl.ds(i * 16, 16)]
      plsc.addupdate_scatter(
          part_v,
          [lax.shift_right_logical(d16, 4), jnp.bitwise_and(d16, 15)], e16)

    plsc.subcore_barrier()
    @pl.loop(0, DROWS // BCH)
    def _(j):
      pltpu.async_copy(part_v.at[pl.ds(j * BCH, BCH)],
                       acc_ref.at[idx_v.at[j]], sem, add=True).wait()
    plsc.subcore_barrier()

    rbase = s * DROWS_PT
    pltpu.sync_copy(acc_ref.at[pl.ds(rbase, DROWS_PT)], tmp_v)

    @pl.loop(0, DROWS_PT)
    def _(r):
      tmp_v[r] = _rsqrt_newton(tmp_v[r] + 1.0)

    pltpu.sync_copy(tmp_v, acc_ref.at[pl.ds(rbase, DROWS_PT)])
    pltpu.sync_copy(tmp_v, dis_hbm.at[pl.ds(rbase, DROWS_PT)])
    plsc.subcore_barrier()
    pltpu.sync_copy(acc_ref, part_v)

    pltpu.sync_copy(src_hbm.at[pl.ds(base, EPT_A)], src_v)

    @plsc.parallel_loop(0, NVEC_A, unroll=4)
    def _(i):
      s16 = src_v[pl.ds(i * 16, 16)]
      d16 = plsc.load_gather(
          part_v,
          [lax.shift_right_logical(s16, 4), jnp.bitwise_and(s16, 15)])
      w_v[pl.ds(i * 16, 16)] = d16 * ew_v[pl.ds(i * 16, 16)]

    pltpu.sync_copy(w_v, w_hbm.at[pl.ds(base, EPT_A)])


_sc_deg = pl.kernel(
    _sc_deg_body,
    out_type=[jax.ShapeDtypeStruct((DROWS, 16), jnp.float32),
              jax.ShapeDtypeStruct((EPAD,), jnp.float32)],
    mesh=_mesh,
    compiler_params=_sc_params,
    scratch_types=[
        pltpu.VMEM((EPT_A,), jnp.int32),
        pltpu.VMEM((EPT_A,), jnp.int32),
        pltpu.VMEM((EPT_A,), jnp.float32),
        pltpu.VMEM((DROWS, 16), jnp.float32),
        pltpu.VMEM((DROWS // BCH, BCH), jnp.int32),
        pltpu.VMEM((DROWS_PT, 16), jnp.float32),
        pltpu.VMEM((EPT_A,), jnp.float32),
        pltpu.VMEM_SHARED((DROWS, 16), jnp.float32),
        pltpu.SemaphoreType.DMA,
    ])


def _sc_scatter_body(h_hbm, src_hbm, dst2_hbm, w_hbm, zrows_hbm,
                     out_hbm,
                     src_v, dst_v, w_v,
                     r0, r1, r2, r3, r4, r5,
                     acc_ref,
                     g0, g1, g2, g3, g4, g5,
                     s0, s1, s2, s3, s4, s5):
  c = lax.axis_index("c")
  s = lax.axis_index("s")
  wid = c * NS + s
  base = wid * EPT_B
  rows = [r0, r1, r2, r3, r4, r5]
  gs = [g0, g1, g2, g3, g4, g5]
  ss = [s0, s1, s2, s3, s4, s5]

  pltpu.sync_copy(src_hbm.at[pl.ds(base, EPT_B)], src_v)
  pltpu.sync_copy(dst2_hbm.at[wid], dst_v)
  pltpu.sync_copy(w_hbm.at[pl.ds(base, EPT_B)], w_v)
  pltpu.sync_copy(zrows_hbm, acc_ref.at[pl.ds(s * ROWS_PT, ROWS_PT)])
  plsc.subcore_barrier()

  def gather_start(ch, b):
    pltpu.async_copy(h_hbm.at[src_v.at[pl.ds(ch * BCH, BCH)]], rows[b], gs[b])

  def gather_wait(b):
    pltpu.make_async_copy(h_hbm.at[pl.ds(0, BCH)], rows[b], gs[b]).wait()

  def scatter_start(ch, b):
    pltpu.async_copy(rows[b], acc_ref.at[dst_v.at[ch]], ss[b], add=True)

  def scatter_wait(b):
    pltpu.make_async_copy(h_hbm.at[pl.ds(0, BCH)], rows[b], ss[b]).wait()

  def scale(ch, b):
    rb = rows[b]

    @plsc.parallel_loop(0, BCH, unroll=8)
    def _(e):
      wv = plsc.load_gather(w_v, [jnp.full((16,), ch * BCH + e, jnp.int32)])
      for q in range(4):
        sl = pl.ds(q * 16, 16)
        rb[e, sl] = rb[e, sl] * wv

  NB, LA = 6, 3
  NMAIN = (NCH_B // NB) * NB - NB
  assert NMAIN % NB == 0 and NMAIN - 1 + LA < NCH_B
  for k in range(LA):
    gather_start(k, k)

  @pl.loop(0, NMAIN // NB)
  def _(j):
    for b in range(NB):
      i = NB * j + b
      bn = (b + LA) % NB
      if b < LA:
        @pl.when(j > 0)
        def _():
          scatter_wait(bn)
      else:
        scatter_wait(bn)
      gather_start(i + LA, bn)
      gather_wait(b)
      scale(i, b)
      scatter_start(i, b)

  for i in range(NMAIN, NCH_B):
    b = i % NB
    bn = (b + LA) % NB
    if i + LA < NCH_B:
      scatter_wait(bn)
      gather_start(i + LA, bn)
    gather_wait(b)
    scale(i, b)
    scatter_start(i, b)
  for i in range(NCH_B - NB, NCH_B):
    scatter_wait(i % NB)

  plsc.subcore_barrier()
  pltpu.sync_copy(acc_ref.at[pl.ds(s * ROWS_PT, ROWS_PT)],
                  out_hbm.at[c, pl.ds(s * ROWS_PT, ROWS_PT)])


_sc_scatter = pl.kernel(
    _sc_scatter_body,
    out_type=jax.ShapeDtypeStruct((NC, NPAD, C), jnp.float32),
    mesh=_mesh,
    compiler_params=_sc_params,
    scratch_types=[
        pltpu.VMEM((EPT_B,), jnp.int32),
        pltpu.VMEM((NCH_B, BCH), jnp.int32),
        pltpu.VMEM((EPT_B,), jnp.float32),
    ] + [pltpu.VMEM((BCH, C), jnp.float32)] * 6
    + [pltpu.VMEM_SHARED((NPAD, C), jnp.float32)]
    + [pltpu.SemaphoreType.DMA] * 12)


def _sc_pairs_body(a_hbm, b_hbm, l0_hbm, l1_hbm,
                   zp_hbm,
                   l0_v, l1_v, ra0, ra1, ra2, ra3, rb0, rb1, rb2, rb3,
                   ga0, ga1, ga2, ga3, gb0, gb1, gb2, gb3,
                   os0, os1, os2, os3):
  c = lax.axis_index("c")
  s = lax.axis_index("s")
  wid = c * NS + s
  base = wid * LPT
  ra = [ra0, ra1, ra2, ra3]
  rb = [rb0, rb1, rb2, rb3]
  ga = [ga0, ga1, ga2, ga3]
  gb = [gb0, gb1, gb2, gb3]
  os_ = [os0, os1, os2, os3]
  pltpu.sync_copy(l0_hbm.at[pl.ds(base, LPT)], l0_v)
  pltpu.sync_copy(l1_hbm.at[pl.ds(base, LPT)], l1_v)

  def gathers_start(ch, b):
    pltpu.async_copy(a_hbm.at[l0_v.at[pl.ds(ch * BCH, BCH)]], ra[b], ga[b])
    pltpu.async_copy(b_hbm.at[l1_v.at[pl.ds(ch * BCH, BCH)]], rb[b], gb[b])

  def gathers_wait(b):
    pltpu.make_async_copy(a_hbm.at[pl.ds(0, BCH)], ra[b], ga[b]).wait()
    pltpu.make_async_copy(b_hbm.at[pl.ds(0, BCH)], rb[b], gb[b]).wait()

  def out_start(ch, b):
    pltpu.async_copy(ra[b], zp_hbm.at[pl.ds(base + ch * BCH, BCH)], os_[b])

  def out_wait(b):
    pltpu.make_async_copy(ra[b], zp_hbm.at[pl.ds(0, BCH)], os_[b]).wait()

  def add(b):
    va, vb = ra[b], rb[b]

    @plsc.parallel_loop(0, BCH, unroll=8)
    def _(r):
      for q in range(4):
        sl = pl.ds(q * 16, 16)
        va[r, sl] = va[r, sl] + vb[r, sl]

  NB, LA = 4, 2
  NMAIN = ((NCH_D - LA) // NB) * NB
  for k in range(LA):
    gathers_start(k, k)

  @pl.loop(0, NMAIN // NB)
  def _(j):
    for b in range(NB):
      i = NB * j + b
      bn = (b + LA) % NB
      if b < LA:
        @pl.when(j > 0)
        def _():
          out_wait(bn)
      else:
        out_wait(bn)
      gathers_start(i + LA, bn)
      gathers_wait(b)
      add(b)
      out_start(i, b)

  for i in range(NMAIN, NCH_D):
    b = i % NB
    bn = (b + LA) % NB
    if i + LA < NCH_D:
      out_wait(bn)
      gathers_start(i + LA, bn)
    gathers_wait(b)
    add(b)
    out_start(i, b)
  for i in range(NCH_D - NB, NCH_D):
    out_wait(i % NB)


_sc_pairs = pl.kernel(
    _sc_pairs_body,
    out_type=jax.ShapeDtypeStruct((NLPAD, C), jnp.float32),
    mesh=_mesh,
    compiler_params=_sc_params,
    scratch_types=[
        pltpu.VMEM((LPT,), jnp.int32),
        pltpu.VMEM((LPT,), jnp.int32),
    ] + [pltpu.VMEM((BCH, C), jnp.float32)] * 8
    + [pltpu.SemaphoreType.DMA] * 12)


_DOT = jnp.dot
_RB = 1000


def _tc_mm1_body(x_ref, w_ref, o_ref):
  o_ref[...] = _DOT(x_ref[...], w_ref[...])


def _tc_mm1(x, w1):
  return pl.pallas_call(
      _tc_mm1_body,
      grid=(N // _RB,),
      in_specs=[pl.BlockSpec((_RB, F_IN), lambda i: (i, 0)),
                pl.BlockSpec((F_IN, C), lambda i: (0, 0))],
      out_specs=pl.BlockSpec((_RB, C), lambda i: (i, 0)),
      out_shape=jax.ShapeDtypeStruct((N, C), jnp.float32),
  )(x, w1)


def _tc_mid_body(acc_ref, hp_ref, dis_ref, b_ref, a_ref, w_ref, o_ref):
  dis = dis_ref[...]
  acc = acc_ref[0] + acc_ref[1]
  pre = (acc + dis * hp_ref[...]) * dis + b_ref[...]
  h = jnp.where(pre >= 0, pre, a_ref[0, 0] * pre)
  o_ref[...] = _DOT(h, w_ref[...])


def _tc_mid(acc, hp, dis, b, a, w):
  return pl.pallas_call(
      _tc_mid_body,
      grid=(N // _RB,),
      in_specs=[pl.BlockSpec((NC, _RB, C), lambda i: (0, i, 0)),
                pl.BlockSpec((_RB, C), lambda i: (i, 0)),
                pl.BlockSpec((_RB, 1), lambda i: (i, 0)),
                pl.BlockSpec((1, C), lambda i: (0, 0)),
                pl.BlockSpec((1, 1), lambda i: (0, 0)),
                pl.BlockSpec((C, C), lambda i: (0, 0))],
      out_specs=pl.BlockSpec((_RB, C), lambda i: (i, 0)),
      out_shape=jax.ShapeDtypeStruct((N, C), jnp.float32),
  )(acc, hp, dis, b, a, w)


def _tc_head_body(acc_ref, hp_ref, dis_ref, b_ref, a_ref, wa_ref, wb_ref,
                  bd_ref, oa_ref, ob_ref):
  dis = dis_ref[...]
  acc = acc_ref[0] + acc_ref[1]
  pre = (acc + dis * hp_ref[...]) * dis + b_ref[...]
  h = jnp.where(pre >= 0, pre, a_ref[0, 0] * pre)
  oa_ref[...] = _DOT(h, wa_ref[...]) + bd_ref[...]
  ob_ref[...] = _DOT(h, wb_ref[...])


def _tc_head(acc, hp, dis, b, a, wa, wb, bd):
  return pl.pallas_call(
      _tc_head_body,
      grid=(N // _RB,),
      in_specs=[pl.BlockSpec((NC, _RB, C), lambda i: (0, i, 0)),
                pl.BlockSpec((_RB, C), lambda i: (i, 0)),
                pl.BlockSpec((_RB, 1), lambda i: (i, 0)),
                pl.BlockSpec((1, C), lambda i: (0, 0)),
                pl.BlockSpec((1, 1), lambda i: (0, 0)),
                pl.BlockSpec((C, C), lambda i: (0, 0)),
                pl.BlockSpec((C, C), lambda i: (0, 0)),
                pl.BlockSpec((1, C), lambda i: (0, 0))],
      out_specs=[pl.BlockSpec((_RB, C), lambda i: (i, 0)),
                 pl.BlockSpec((_RB, C), lambda i: (i, 0))],
      out_shape=[jax.ShapeDtypeStruct((N, C), jnp.float32),
                 jax.ShapeDtypeStruct((N, C), jnp.float32)],
  )(acc, hp, dis, b, a, wa, wb, bd)


_LB = 6400


def _tc_out_body(zp_ref, a_ref, w_ref, b_ref, o_ref):
  z = zp_ref[...]
  z = jnp.where(z >= 0, z, a_ref[0, 0] * z)
  o_ref[...] = _DOT(z, w_ref[...]) + b_ref[0, 0]


def _tc_out(zp, a, w, b):
  return pl.pallas_call(
      _tc_out_body,
      grid=(NLPAD // _LB,),
      in_specs=[pl.BlockSpec((_LB, C), lambda i: (i, 0)),
                pl.BlockSpec((1, 1), lambda i: (0, 0)),
                pl.BlockSpec((C, 1), lambda i: (0, 0)),
                pl.BlockSpec((1, 1), lambda i: (0, 0))],
      out_specs=pl.BlockSpec((_LB, 1), lambda i: (i, 0)),
      out_shape=jax.ShapeDtypeStruct((NLPAD, 1), jnp.float32),
  )(zp, a, w, b)


def kernel(x, edge_index, edge_weight, label_edge_index,
           W1, b1, a1, W2, b2, a2, Wd1, bd1, ad, Wd2, bd2):
  i32 = jnp.int32
  f32 = jnp.float32
  src = edge_index[0].astype(i32)
  dst = edge_index[1].astype(i32)
  ew = edge_weight.astype(f32)
  epad = EPAD - E
  src_p = jnp.concatenate([src, jnp.zeros((epad,), i32)])
  dst_p = jnp.concatenate([dst, jnp.zeros((epad,), i32)])
  ew_p = jnp.concatenate([ew, jnp.zeros((epad,), f32)])
  lpad = NLPAD - NL
  l0_p = jnp.concatenate([label_edge_index[0].astype(i32),
                          jnp.zeros((lpad,), i32)])
  l1_p = jnp.concatenate([label_edge_index[1].astype(i32),
                          jnp.zeros((lpad,), i32)])

  zdeg = jnp.zeros((DROWS, 16), f32)
  iota_rows = jnp.arange(DROWS, dtype=i32).reshape(DROWS // BCH, BCH)
  zrows = jnp.zeros((ROWS_PT, C), f32)
  dst2 = dst_p.reshape(NW, NCH_B, BCH)

  dis2d, w_e = _sc_deg(src_p, dst_p, ew_p, zdeg, iota_rows)
  dis = dis2d.reshape(NPAD)[:N].reshape(N, 1)

  h1p = _tc_mm1(x, W1)
  acc1 = _sc_scatter(h1p, src_p, dst2, w_e, zrows)
  h2p = _tc_mid(acc1, h1p, dis, b1.reshape(1, C), a1.reshape(1, 1), W2)

  acc2 = _sc_scatter(h2p, src_p, dst2, w_e, zrows)
  A, B = _tc_head(acc2, h2p, dis, b2.reshape(1, C), a2.reshape(1, 1),
                  Wd1[:C], Wd1[C:], bd1.reshape(1, C))

  zp = _sc_pairs(A, B, l0_p, l1_p)
  out = _tc_out(zp, ad.reshape(1, 1), Wd2, bd2.reshape(1, 1))
  return out[:NL]
